# Initial kernel scaffold; baseline (speedup 1.0000x reference)
import jax
import jax.numpy as jnp
from jax.experimental import pallas as pl


def kernel(x, adj_indices, adj_values, weight, bias):
    raise NotImplementedError("write your pallas kernel here")


# speedup vs baseline: 3.2576x; 3.2576x over previous
import functools

import jax
import jax.numpy as jnp
from jax import lax
from jax.experimental import pallas as pl
from jax.experimental.pallas import tpu as pltpu
import jax.experimental.pallas.tpu_sc as plsc

N_NODES = 100000
N_EDGES = 1600000
D = 128

NC = 2
NS = 16
L = 16

R = 10240
PASSES = 5
EPT = N_EDGES // NS
C = 2000
NCHUNK = EPT // C
G = 64
GSH = 6
NB = 33
RPT = R // NS
CPR = 160
ZR = 40

BM = 2000


def _mm_body(x_ref, w_ref, b_ref, h_ref):
    h_ref[...] = (
        jnp.dot(x_ref[...], w_ref[...], preferred_element_type=jnp.float32)
        + b_ref[...]
    )


def _matmul(x, w, b2d):
    return pl.pallas_call(
        _mm_body,
        grid=(N_NODES // BM,),
        in_specs=[
            pl.BlockSpec((BM, D), lambda i: (i, 0)),
            pl.BlockSpec((D, D), lambda i: (0, 0)),
            pl.BlockSpec((1, D), lambda i: (0, 0)),
        ],
        out_specs=pl.BlockSpec((BM, D), lambda i: (i, 0)),
        out_shape=jax.ShapeDtypeStruct((N_NODES, D), jnp.float32),
    )(x, w, b2d)


_mesh = plsc.VectorSubcoreMesh(
    core_axis_name="c", subcore_axis_name="s", num_cores=NC, num_subcores=NS
)


@functools.partial(
    pl.kernel,
    out_type=jax.ShapeDtypeStruct((N_NODES, D), jnp.float32),
    mesh=_mesh,
    scratch_types=[
        pltpu.VMEM((C,), jnp.int32),
        pltpu.VMEM((C,), jnp.int32),
        pltpu.VMEM((C,), jnp.float32),
        pltpu.VMEM((NB, G), jnp.int32),
        pltpu.VMEM((NB, G), jnp.int32),
        pltpu.VMEM((NB, G), jnp.float32),
        pltpu.VMEM((G, D), jnp.float32),
        pltpu.VMEM((ZR, D), jnp.float32),
        pltpu.VMEM_SHARED((R, D), jnp.float32),
        pltpu.SemaphoreType.DMA,
    ],
    compiler_params=pltpu.CompilerParams(needs_layout_passes=False),
)
def _aggregate(h_hbm, rows_hbm, cols_hbm, vals_hbm, out_hbm,
               rbuf, cbuf, vbuf, cidx, clrow, cval, gbuf, zbuf, acc, sem):
    s = lax.axis_index("s")
    c = lax.axis_index("c")
    zero16f = jnp.zeros((L,), jnp.float32)
    zero16i = jnp.zeros((L,), jnp.int32)
    iota = lax.iota(jnp.int32, L)

    def zinit(i, carry):
        zbuf[i >> 3, pl.ds((i & 7) * L, L)] = zero16f
        return carry

    lax.fori_loop(0, ZR * (D // L), zinit, 0)

    for p in range(PASSES):
        base = (NC * p + c) * R

        for z in range(RPT // ZR):
            pltpu.sync_copy(zbuf, acc.at[pl.ds(s * RPT + z * ZR, ZR)])

        plsc.subcore_barrier()

        if True:
            lo = base
            hi = jnp.minimum(base + R, N_NODES)
            ebase = s * EPT

            def chunk_body(ch, carry):
                e0 = ebase + ch * C
                cp1 = pltpu.async_copy(rows_hbm.at[pl.ds(e0, C)], rbuf, sem)
                cp2 = pltpu.async_copy(cols_hbm.at[pl.ds(e0, C)], cbuf, sem)
                cp3 = pltpu.async_copy(vals_hbm.at[pl.ds(e0, C)], vbuf, sem)
                cp1.wait()
                cp2.wait()
                cp3.wait()

                def step(i, off):
                    r = rbuf[pl.ds(i * L, L)]
                    col = cbuf[pl.ds(i * L, L)]
                    v = vbuf[pl.ds(i * L, L)]
                    m = (r >= lo) & (r < hi)
                    pre = plsc.cumsum(jnp.where(m, 1, 0).astype(jnp.int32))
                    flat = jnp.maximum(off + pre - 1, 0)
                    ir = flat >> GSH
                    ic = flat & (G - 1)
                    plsc.store_scatter(cidx, [ir, ic], col, mask=m)
                    plsc.store_scatter(clrow, [ir, ic], r - lo, mask=m)
                    plsc.store_scatter(cval, [ir, ic], v, mask=m)
                    return off + plsc.all_reduce_population_count(m)

                off = lax.fori_loop(0, C // L, step, zero16i)
                n = off[0]

                for kp in range(G // L):
                    flatp = n + kp * L + iota
                    irp = flatp >> GSH
                    icp = flatp & (G - 1)
                    spread = (s * 64 + ch) * L + iota
                    plsc.store_scatter(cidx, [irp, icp], spread)
                    plsc.store_scatter(clrow, [irp, icp], zero16i)
                    plsc.store_scatter(cval, [irp, icp], zero16f)

                nb = (n + G - 1) // G

                def gather_body(j, carry2):
                    pltpu.async_copy(h_hbm.at[cidx.at[j]], gbuf, sem).wait()
                    jfull = jnp.full((L,), j, jnp.int32)

                    def mul_body(e, carry3):
                        vv = plsc.load_gather(
                            cval, [jfull, jnp.full((L,), e, jnp.int32)]
                        )
                        for f in range(D // L):
                            gbuf[e, pl.ds(f * L, L)] = gbuf[e, pl.ds(f * L, L)] * vv
                        return carry3

                    lax.fori_loop(0, G, mul_body, 0)
                    pltpu.sync_copy(gbuf, acc.at[clrow.at[j]], add=True)
                    return carry2

                lax.fori_loop(0, nb, gather_body, 0)
                return carry

            lax.fori_loop(0, NCHUNK, chunk_body, 0)

        plsc.subcore_barrier()

        for q in range(RPT // CPR):
            off = s * RPT + q * CPR

            @pl.when(base + off < N_NODES)
            def _copy_out(off=off):
                pltpu.sync_copy(
                    acc.at[pl.ds(off, CPR)],
                    out_hbm.at[pl.ds(base + off, CPR)],
                )

        plsc.subcore_barrier()


def kernel(x, adj_indices, adj_values, weight, bias):
    rows = adj_indices[0].astype(jnp.int32)
    cols = adj_indices[1].astype(jnp.int32)
    h = _matmul(x, weight, bias.reshape(1, D))
    return _aggregate(h, rows, cols, adj_values)
